# Initial kernel scaffold; baseline (speedup 1.0000x reference)
import jax
import jax.numpy as jnp
from jax.experimental import pallas as pl


def kernel(pitch_indices, velocity_indices, program_indices, continuous_features, drum_indices, W_pitch, W_velocity, W_program, W_drum):
    raise NotImplementedError("write your pallas kernel here")


# speedup vs baseline: 60.1830x; 60.1830x over previous
import jax
import jax.numpy as jnp
from jax.experimental import pallas as pl
from jax.experimental.pallas import tpu as pltpu

_EMBED = 512
_N = 16384
_R = 128


def _combine_body(pitch_ref, vel_ref, prog_ref, drum_ref, cont_ref,
                  wp_ref, wv_ref, wg_ref, wd_ref, out_ref):
    f32 = jnp.float32

    def counts_of(idx_ref):
        idx2 = idx_ref[...]
        bins = jax.lax.broadcasted_iota(jnp.int32, (128, _R, _R), 0)
        cmp = (bins == idx2[None, :, :]).astype(f32)
        c = jnp.sum(cmp, axis=(1, 2))
        return c.reshape(1, 128)

    def bag(counts, w_ref):
        return jax.lax.dot_general(
            counts, w_ref[...], (((1,), (0,)), ((), ())),
            precision=jax.lax.Precision.HIGHEST,
            preferred_element_type=f32)

    pitch_bag = bag(counts_of(pitch_ref), wp_ref)
    vel_bag = bag(counts_of(vel_ref), wv_ref)
    prog_bag = bag(counts_of(prog_ref), wg_ref)

    c1 = jnp.sum(drum_ref[...].astype(f32))
    drum_bag = (_N - c1) * wd_ref[0:1, :] + c1 * wd_ref[1:2, :]

    i2 = jax.lax.broadcasted_iota(jnp.int32, (1, 256), 1).astype(f32)
    freqs = jnp.exp((2.0 * i2 / _EMBED) * jnp.log(10000.0).astype(f32))
    t0 = cont_ref[0, 0] * freqs
    t1 = cont_ref[0, 1] * freqs
    time0 = jnp.concatenate([jnp.sin(t0), jnp.cos(t0)], axis=1)
    time1 = jnp.concatenate([jnp.sin(t1), jnp.cos(t1)], axis=1)

    out_ref[:, 0:512] = drum_bag
    out_ref[:, 512:1024] = time0
    out_ref[:, 1024:1536] = time1
    out_ref[:, 1536:2048] = prog_bag
    out_ref[:, 2048:2560] = pitch_bag
    out_ref[:, 2560:3072] = vel_bag


def kernel(pitch_indices, velocity_indices, program_indices,
           continuous_features, drum_indices,
           W_pitch, W_velocity, W_program, W_drum):
    p2 = pitch_indices.reshape(_R, _R)
    v2 = velocity_indices.reshape(_R, _R)
    g2 = program_indices.reshape(_R, _R)
    d2 = drum_indices.reshape(_R, _R)
    cont = continuous_features.reshape(1, 2)
    out = pl.pallas_call(
        _combine_body,
        out_shape=jax.ShapeDtypeStruct((1, 6 * _EMBED), jnp.float32),
    )(p2, v2, g2, d2, cont, W_pitch, W_velocity, W_program, W_drum)
    return out
